# Initial kernel scaffold; baseline (speedup 1.0000x reference)
import jax
import jax.numpy as jnp
from jax.experimental import pallas as pl


def kernel(X, edge_index_0, edge_weight_0, edge_index_1, edge_weight_1, res_n_id_0, res_n_id_1, size0_dst, size1_dst, w_n1, w_e1, q1, k1, att_w1, att_b1, cat_w1, cat_b1, w_n2, w_e2, q2, k2, att_w2, att_b2, cat_w2, cat_b2):
    raise NotImplementedError("write your pallas kernel here")


# speedup vs baseline: 34.2665x; 34.2665x over previous
import functools

import jax
import jax.numpy as jnp
from jax import lax
from jax.experimental import pallas as pl
from jax.experimental.pallas import tpu as pltpu
from jax.experimental.pallas import tpu_sc as plsc

_N1, _N2, _N3 = 50000, 25000, 12500
_E1, _E2 = 800000, 400000

_NW = 32
_N2P = 25088
_N3P = 12544
_E1PW, _E2PW = 25088, 12544
_ECH = 128

_MESH = plsc.VectorSubcoreMesh(
    core_axis_name="c", subcore_axis_name="s", num_cores=2, num_subcores=16)
_SC_PARAMS = pltpu.CompilerParams(
    use_tc_tiling_on_sc=False, needs_layout_passes=False)


def _sigmoid(x):
    return 1.0 / (1.0 + jnp.exp(-x))


def _tc_proj1(x0, x1, w0, w1, bias):
    rb = 400

    def body(x0_ref, x1_ref, w0_ref, w1_ref, b_ref, o_ref):
        o_ref[...] = (
            jnp.dot(x0_ref[...], w0_ref[...], preferred_element_type=jnp.float32)
            + jnp.dot(x1_ref[...], w1_ref[...], preferred_element_type=jnp.float32)
            + b_ref[...])

    return pl.pallas_call(
        body,
        grid=(_N1 // rb,),
        in_specs=[
            pl.BlockSpec((rb, 32), lambda i: (i, 0)),
            pl.BlockSpec((rb, 32), lambda i: (i, 0)),
            pl.BlockSpec((32, 48), lambda i: (0, 0)),
            pl.BlockSpec((32, 48), lambda i: (0, 0)),
            pl.BlockSpec((1, 48), lambda i: (0, 0)),
        ],
        out_specs=pl.BlockSpec((rb, 48), lambda i: (i, 0)),
        out_shape=jax.ShapeDtypeStruct((_N1, 48), jnp.float32),
    )(x0, x1, w0, w1, bias)


def _tc_update1(xdf, p0, p1, ca, cb, cbias, m0, m1, b2):
    rb = 256

    def body(xdf_ref, p0_ref, p1_ref, ca_ref, cb_ref, cbias_ref,
             m0_ref, m1_ref, b2_ref, o_ref):
        xd0 = xdf_ref[:, 0:16]
        xd1 = xdf_ref[:, 16:32]
        a0 = p0_ref[:, 0:16] + p1_ref[:, 0:16]
        a1 = p0_ref[:, 16:32] + p1_ref[:, 16:32]
        o0 = (jnp.dot(xd0, ca_ref[...], preferred_element_type=jnp.float32)
              + jnp.dot(a0, cb_ref[...], preferred_element_type=jnp.float32)
              + cbias_ref[...])
        o1 = (jnp.dot(xd1, ca_ref[...], preferred_element_type=jnp.float32)
              + jnp.dot(a1, cb_ref[...], preferred_element_type=jnp.float32)
              + cbias_ref[...])
        s = jnp.sum(o0, axis=1, keepdims=True) + jnp.sum(o1, axis=1, keepdims=True)
        m = s * (1.0 / 32.0)
        ss = (jnp.sum(o0 * o0, axis=1, keepdims=True)
              + jnp.sum(o1 * o1, axis=1, keepdims=True))
        var = (ss - 32.0 * m * m) * (1.0 / 31.0)
        inv = lax.rsqrt(var + 1e-5)
        h0 = xd0 + (o0 - m) * inv
        h1 = xd1 + (o1 - m) * inv
        h0 = jnp.where(h0 >= 0, h0, 0.01 * h0)
        h1 = jnp.where(h1 >= 0, h1, 0.01 * h1)
        o_ref[...] = (
            jnp.dot(h0, m0_ref[...], preferred_element_type=jnp.float32)
            + jnp.dot(h1, m1_ref[...], preferred_element_type=jnp.float32)
            + b2_ref[...])

    return pl.pallas_call(
        body,
        grid=(_N2P // rb,),
        in_specs=[
            pl.BlockSpec((rb, 32), lambda i: (i, 0)),
            pl.BlockSpec((rb, 32), lambda i: (i, 0)),
            pl.BlockSpec((rb, 32), lambda i: (i, 0)),
            pl.BlockSpec((16, 16), lambda i: (0, 0)),
            pl.BlockSpec((16, 16), lambda i: (0, 0)),
            pl.BlockSpec((1, 16), lambda i: (0, 0)),
            pl.BlockSpec((16, 80), lambda i: (0, 0)),
            pl.BlockSpec((16, 80), lambda i: (0, 0)),
            pl.BlockSpec((1, 80), lambda i: (0, 0)),
        ],
        out_specs=pl.BlockSpec((rb, 80), lambda i: (i, 0)),
        out_shape=jax.ShapeDtypeStruct((_N2P, 80), jnp.float32),
    )(xdf, p0, p1, ca, cb, cbias, m0, m1, b2)


def _tc_final(xdf, p0, p1, ca, cb, cbias):
    rb = 256

    def body(xdf_ref, p0_ref, p1_ref, ca_ref, cb_ref, cbias_ref, o_ref):
        xd0 = xdf_ref[:, 0:32]
        xd1 = xdf_ref[:, 32:64]
        a0 = p0_ref[:, 0:32] + p1_ref[:, 0:32]
        a1 = p0_ref[:, 32:64] + p1_ref[:, 32:64]
        o0 = (jnp.dot(xd0, ca_ref[...], preferred_element_type=jnp.float32)
              + jnp.dot(a0, cb_ref[...], preferred_element_type=jnp.float32)
              + cbias_ref[...])
        o1 = (jnp.dot(xd1, ca_ref[...], preferred_element_type=jnp.float32)
              + jnp.dot(a1, cb_ref[...], preferred_element_type=jnp.float32)
              + cbias_ref[...])
        s = jnp.sum(o0, axis=1, keepdims=True) + jnp.sum(o1, axis=1, keepdims=True)
        m = s * (1.0 / 64.0)
        ss = (jnp.sum(o0 * o0, axis=1, keepdims=True)
              + jnp.sum(o1 * o1, axis=1, keepdims=True))
        var = (ss - 64.0 * m * m) * (1.0 / 63.0)
        inv = lax.rsqrt(var + 1e-5)
        h0 = xd0 + (o0 - m) * inv
        h1 = xd1 + (o1 - m) * inv
        o_ref[0, :, :] = jnp.where(h0 >= 0, h0, 0.01 * h0)
        o_ref[1, :, :] = jnp.where(h1 >= 0, h1, 0.01 * h1)

    return pl.pallas_call(
        body,
        grid=(_N3P // rb,),
        in_specs=[
            pl.BlockSpec((rb, 64), lambda i: (i, 0)),
            pl.BlockSpec((rb, 64), lambda i: (i, 0)),
            pl.BlockSpec((rb, 64), lambda i: (i, 0)),
            pl.BlockSpec((32, 32), lambda i: (0, 0)),
            pl.BlockSpec((32, 32), lambda i: (0, 0)),
            pl.BlockSpec((1, 32), lambda i: (0, 0)),
        ],
        out_specs=pl.BlockSpec((2, rb, 32), lambda i: (0, i, 0)),
        out_shape=jax.ShapeDtypeStruct((2, _N3P, 32), jnp.float32),
    )(xdf, p0, p1, ca, cb, cbias)


def _sc_resgather(table, res_pad, f, fw, ndstp, pw, ch):
    niter = pw // ch

    @functools.partial(
        pl.kernel,
        out_type=(jax.ShapeDtypeStruct((ndstp, fw), jnp.float32),
                  jax.ShapeDtypeStruct((ndstp, 16), jnp.float32)),
        mesh=_MESH,
        scratch_types=[
            pltpu.VMEM((ch,), jnp.int32),
            pltpu.VMEM((ch, f), jnp.float32),
            pltpu.VMEM((ch, fw), jnp.float32),
            pltpu.VMEM((ch, 16), jnp.float32),
            pltpu.SemaphoreType.DMA,
        ],
        compiler_params=_SC_PARAMS,
    )
    def k(table_ref, res_ref, feats_out, scal_out, idx_v, rows_v, fv, sv, sem):
        wid = lax.axis_index("s") * 2 + lax.axis_index("c")

        def chunk(j, carry):
            base = wid * pw + j * ch
            pltpu.sync_copy(res_ref.at[pl.ds(base, ch)], idx_v)
            pltpu.async_copy(table_ref.at[idx_v], rows_v, sem).wait()

            def row(e, c2):
                for kk in range(fw // 16):
                    fv[e, pl.ds(kk * 16, 16)] = rows_v[e, pl.ds(kk * 16, 16)]
                sv[e, pl.ds(0, 16)] = rows_v[e, pl.ds(fw, 16)]
                return c2

            lax.fori_loop(0, ch, row, 0)
            pltpu.sync_copy(fv, feats_out.at[pl.ds(base, ch)])
            pltpu.sync_copy(sv, scal_out.at[pl.ds(base, ch)])
            return carry

        lax.fori_loop(0, niter, chunk, 0)

    return k(table, res_pad)


def _sc_edges(table, dscal, srcp, dstp, wp, consts, f, fw, c, ndstp, epw, zch):
    mw = 2 * c
    niter = epw // _ECH
    pt = ndstp // 16
    nz = pt // zch

    @functools.partial(
        pl.kernel,
        out_type=jax.ShapeDtypeStruct((2 * ndstp, mw), jnp.float32),
        mesh=_MESH,
        scratch_types=[
            pltpu.VMEM((_ECH,), jnp.int32),
            pltpu.VMEM((_ECH,), jnp.int32),
            pltpu.VMEM((_ECH,), jnp.float32),
            pltpu.VMEM((_ECH, f), jnp.float32),
            pltpu.VMEM((_ECH, 16), jnp.float32),
            pltpu.VMEM((_ECH, mw), jnp.float32),
            pltpu.VMEM((4, 16), jnp.float32),
            pltpu.VMEM((zch, mw), jnp.float32),
            pltpu.VMEM_SHARED((ndstp, mw), jnp.float32),
            pltpu.SemaphoreType.DMA,
            pltpu.SemaphoreType.DMA,
        ],
        compiler_params=_SC_PARAMS,
    )
    def k(table_ref, dscal_ref, src_ref, dst_ref, w_ref, c_ref, part_out,
          sidx, didx, wv, rows, drows, msg, cv, zbuf, aggr, sem1, sem2):
        cid = lax.axis_index("c")
        sid = lax.axis_index("s")
        wid = sid * 2 + cid
        pltpu.sync_copy(c_ref, cv)

        def col16(v):
            return jnp.full((16,), v, jnp.int32)

        zero16 = jnp.zeros((16,), jnp.float32)

        def zrow(e, carry):
            for kk in range(mw // 16):
                zbuf[e, pl.ds(kk * 16, 16)] = zero16
            return carry

        lax.fori_loop(0, zch, zrow, 0)
        for j in range(nz):
            pltpu.sync_copy(zbuf, aggr.at[pl.ds(sid * pt + j * zch, zch)])
        plsc.subcore_barrier()

        ce = cv[2, pl.ds(0, 16)][0]
        wrows = [cv[kk, pl.ds(0, 16)] for kk in range(c // 16)]
        iot = lax.iota(jnp.int32, 16)
        colq0 = col16(fw)
        colq1 = col16(fw + 1)
        colk0 = col16(2)
        colk1 = col16(3)

        def chunk(j, carry):
            base = wid * epw + j * _ECH
            pltpu.sync_copy(src_ref.at[pl.ds(base, _ECH)], sidx)
            pltpu.sync_copy(dst_ref.at[pl.ds(base, _ECH)], didx)
            pltpu.sync_copy(w_ref.at[pl.ds(base, _ECH)], wv)
            cp1 = pltpu.async_copy(table_ref.at[sidx], rows, sem1)
            cp2 = pltpu.async_copy(dscal_ref.at[didx], drows, sem2)
            cp1.wait()
            cp2.wait()

            def group(g, c2):
                rid = iot + g * 16
                sq0 = plsc.load_gather(rows, [rid, colq0])
                sq1 = plsc.load_gather(rows, [rid, colq1])
                sk0 = plsc.load_gather(drows, [rid, colk0])
                sk1 = plsc.load_gather(drows, [rid, colk1])
                wg = wv[pl.ds(g * 16, 16)]
                cwg = ce * wg
                a0 = _sigmoid(sq0 + sk0 + cwg)
                a1 = _sigmoid(sq1 + sk1 + cwg)
                for kk in range(c // 16):
                    wrow = wrows[kk]
                    for cc in range(16):
                        col = kk * 16 + cc
                        gcol = _sigmoid(wg * wrow[cc])
                        x0 = plsc.load_gather(rows, [rid, col16(col)])
                        x1 = plsc.load_gather(rows, [rid, col16(c + col)])
                        plsc.store_scatter(msg, [rid, col16(col)], (a0 * gcol) * x0)
                        plsc.store_scatter(msg, [rid, col16(c + col)], (a1 * gcol) * x1)
                return c2

            lax.fori_loop(0, _ECH // 16, group, 0)
            pltpu.sync_copy(msg, aggr.at[didx], add=True)
            return carry

        lax.fori_loop(0, niter, chunk, 0)
        plsc.subcore_barrier()

        for j in range(nz):
            off = sid * pt + j * zch
            pltpu.sync_copy(aggr.at[pl.ds(off, zch)], zbuf)
            pltpu.sync_copy(zbuf, part_out.at[pl.ds(cid * ndstp + off, zch)])

    return k(table, dscal, srcp, dstp, wp, consts)


def kernel(X, edge_index_0, edge_weight_0, edge_index_1, edge_weight_1,
           res_n_id_0, res_n_id_1, size0_dst, size1_dst,
           w_n1, w_e1, q1, k1, att_w1, att_b1, cat_w1, cat_b1,
           w_n2, w_e2, q2, k2, att_w2, att_b2, cat_w2, cat_b2):
    f32 = jnp.float32

    vq1 = w_n1 @ (q1 @ att_w1[0:16, 0])
    vk1 = w_n1 @ (k1 @ att_w1[16:32, 0])
    ce1 = w_e1[0] @ att_w1[32:48, 0]
    w0 = (jnp.zeros((32, 48), f32)
          .at[:, 0:16].set(w_n1).at[:, 32].set(vq1).at[:, 34].set(vk1))
    w1 = (jnp.zeros((32, 48), f32)
          .at[:, 16:32].set(w_n1).at[:, 33].set(vq1).at[:, 35].set(vk1))
    bias1 = jnp.zeros((1, 48), f32).at[0, 34].set(att_b1[0]).at[0, 35].set(att_b1[0])

    xs1 = _tc_proj1(X[0], X[1], w0, w1, bias1)

    res0p = jnp.pad(res_n_id_0.astype(jnp.int32), (0, _N2P - _N2))
    xd1, dscal1 = _sc_resgather(xs1, res0p, 48, 32, _N2P, 784, 112)

    pad_e1 = _NW * _E1PW - _E1
    src0p = jnp.pad(edge_index_0[0].astype(jnp.int32), (0, pad_e1))
    dst0p = jnp.pad(edge_index_0[1].astype(jnp.int32), (0, pad_e1),
                    constant_values=_N2P - 1)
    ew0p = jnp.pad(edge_weight_0, (0, pad_e1))
    consts1 = jnp.zeros((4, 16), f32).at[0].set(w_e1[0]).at[2, 0].set(ce1)
    part1 = _sc_edges(xs1, dscal1, src0p, dst0p, ew0p, consts1,
                      48, 32, 16, _N2P, _E1PW, 112)

    vq2 = w_n2 @ (q2 @ att_w2[0:32, 0])
    vk2 = w_n2 @ (k2 @ att_w2[32:64, 0])
    ce2 = w_e2[0] @ att_w2[64:96, 0]
    m0 = (jnp.zeros((16, 80), f32)
          .at[:, 0:32].set(w_n2).at[:, 64].set(vq2).at[:, 66].set(vk2))
    m1 = (jnp.zeros((16, 80), f32)
          .at[:, 32:64].set(w_n2).at[:, 65].set(vq2).at[:, 67].set(vk2))
    bias2 = jnp.zeros((1, 80), f32).at[0, 66].set(att_b2[0]).at[0, 67].set(att_b2[0])

    xs2 = _tc_update1(xd1, part1[0:_N2P], part1[_N2P:2 * _N2P],
                      cat_w1[0:16], cat_w1[16:32], cat_b1[None, :],
                      m0, m1, bias2)

    res1p = jnp.pad(res_n_id_1.astype(jnp.int32), (0, _N3P - _N3))
    xd2, dscal2 = _sc_resgather(xs2, res1p, 80, 64, _N3P, 392, 56)

    pad_e2 = _NW * _E2PW - _E2
    src1p = jnp.pad(edge_index_1[0].astype(jnp.int32), (0, pad_e2))
    dst1p = jnp.pad(edge_index_1[1].astype(jnp.int32), (0, pad_e2),
                    constant_values=_N3P - 1)
    ew1p = jnp.pad(edge_weight_1, (0, pad_e2))
    consts2 = (jnp.zeros((4, 16), f32)
               .at[0].set(w_e2[0, 0:16]).at[1].set(w_e2[0, 16:32])
               .at[2, 0].set(ce2))
    part2 = _sc_edges(xs2, dscal2, src1p, dst1p, ew1p, consts2,
                      80, 64, 32, _N3P, _E2PW, 56)

    outp = _tc_final(xd2, part2[0:_N3P], part2[_N3P:2 * _N3P],
                     cat_w2[0:32], cat_w2[32:64], cat_b2[None, :])
    return outp[:, 0:_N3, :]


# speedup vs baseline: 43.6695x; 1.2744x over previous
import functools

import jax
import jax.numpy as jnp
from jax import lax
from jax.experimental import pallas as pl
from jax.experimental.pallas import tpu as pltpu
from jax.experimental.pallas import tpu_sc as plsc

_N1, _N2, _N3 = 50000, 25000, 12500
_E1, _E2 = 800000, 400000

_NW = 32
_N2P = 25088
_N3P = 12544
_E1PW, _E2PW = 25088, 12544
_ECH = 128

_MESH = plsc.VectorSubcoreMesh(
    core_axis_name="c", subcore_axis_name="s", num_cores=2, num_subcores=16)
_SC_PARAMS = pltpu.CompilerParams(
    use_tc_tiling_on_sc=False, needs_layout_passes=False)


def _sigmoid(x):
    return 1.0 / (1.0 + jnp.exp(-x))


def _tc_proj1(x0, x1, w0, w1, bias):
    rb = 400

    def body(x0_ref, x1_ref, w0_ref, w1_ref, b_ref, o_ref):
        o_ref[...] = (
            jnp.dot(x0_ref[...], w0_ref[...], preferred_element_type=jnp.float32)
            + jnp.dot(x1_ref[...], w1_ref[...], preferred_element_type=jnp.float32)
            + b_ref[...])

    return pl.pallas_call(
        body,
        grid=(_N1 // rb,),
        in_specs=[
            pl.BlockSpec((rb, 32), lambda i: (i, 0)),
            pl.BlockSpec((rb, 32), lambda i: (i, 0)),
            pl.BlockSpec((32, 48), lambda i: (0, 0)),
            pl.BlockSpec((32, 48), lambda i: (0, 0)),
            pl.BlockSpec((1, 48), lambda i: (0, 0)),
        ],
        out_specs=pl.BlockSpec((rb, 48), lambda i: (i, 0)),
        out_shape=jax.ShapeDtypeStruct((_N1, 48), jnp.float32),
    )(x0, x1, w0, w1, bias)


def _tc_update1(xdf, p0, p1, ca, cb, cbias, m0, m1, b2):
    rb = 256

    def body(xdf_ref, p0_ref, p1_ref, ca_ref, cb_ref, cbias_ref,
             m0_ref, m1_ref, b2_ref, o_ref):
        xd0 = xdf_ref[:, 0:16]
        xd1 = xdf_ref[:, 16:32]
        a0 = p0_ref[:, 0:16] + p1_ref[:, 0:16]
        a1 = p0_ref[:, 16:32] + p1_ref[:, 16:32]
        o0 = (jnp.dot(xd0, ca_ref[...], preferred_element_type=jnp.float32)
              + jnp.dot(a0, cb_ref[...], preferred_element_type=jnp.float32)
              + cbias_ref[...])
        o1 = (jnp.dot(xd1, ca_ref[...], preferred_element_type=jnp.float32)
              + jnp.dot(a1, cb_ref[...], preferred_element_type=jnp.float32)
              + cbias_ref[...])
        s = jnp.sum(o0, axis=1, keepdims=True) + jnp.sum(o1, axis=1, keepdims=True)
        m = s * (1.0 / 32.0)
        ss = (jnp.sum(o0 * o0, axis=1, keepdims=True)
              + jnp.sum(o1 * o1, axis=1, keepdims=True))
        var = (ss - 32.0 * m * m) * (1.0 / 31.0)
        inv = lax.rsqrt(var + 1e-5)
        h0 = xd0 + (o0 - m) * inv
        h1 = xd1 + (o1 - m) * inv
        h0 = jnp.where(h0 >= 0, h0, 0.01 * h0)
        h1 = jnp.where(h1 >= 0, h1, 0.01 * h1)
        o_ref[...] = (
            jnp.dot(h0, m0_ref[...], preferred_element_type=jnp.float32)
            + jnp.dot(h1, m1_ref[...], preferred_element_type=jnp.float32)
            + b2_ref[...])

    return pl.pallas_call(
        body,
        grid=(_N2P // rb,),
        in_specs=[
            pl.BlockSpec((rb, 32), lambda i: (i, 0)),
            pl.BlockSpec((rb, 32), lambda i: (i, 0)),
            pl.BlockSpec((rb, 32), lambda i: (i, 0)),
            pl.BlockSpec((16, 16), lambda i: (0, 0)),
            pl.BlockSpec((16, 16), lambda i: (0, 0)),
            pl.BlockSpec((1, 16), lambda i: (0, 0)),
            pl.BlockSpec((16, 80), lambda i: (0, 0)),
            pl.BlockSpec((16, 80), lambda i: (0, 0)),
            pl.BlockSpec((1, 80), lambda i: (0, 0)),
        ],
        out_specs=pl.BlockSpec((rb, 80), lambda i: (i, 0)),
        out_shape=jax.ShapeDtypeStruct((_N2P, 80), jnp.float32),
    )(xdf, p0, p1, ca, cb, cbias, m0, m1, b2)


def _tc_final(xdf, p0, p1, ca, cb, cbias):
    rb = 256

    def body(xdf_ref, p0_ref, p1_ref, ca_ref, cb_ref, cbias_ref, o_ref):
        xd0 = xdf_ref[:, 0:32]
        xd1 = xdf_ref[:, 32:64]
        a0 = p0_ref[:, 0:32] + p1_ref[:, 0:32]
        a1 = p0_ref[:, 32:64] + p1_ref[:, 32:64]
        o0 = (jnp.dot(xd0, ca_ref[...], preferred_element_type=jnp.float32)
              + jnp.dot(a0, cb_ref[...], preferred_element_type=jnp.float32)
              + cbias_ref[...])
        o1 = (jnp.dot(xd1, ca_ref[...], preferred_element_type=jnp.float32)
              + jnp.dot(a1, cb_ref[...], preferred_element_type=jnp.float32)
              + cbias_ref[...])
        s = jnp.sum(o0, axis=1, keepdims=True) + jnp.sum(o1, axis=1, keepdims=True)
        m = s * (1.0 / 64.0)
        ss = (jnp.sum(o0 * o0, axis=1, keepdims=True)
              + jnp.sum(o1 * o1, axis=1, keepdims=True))
        var = (ss - 64.0 * m * m) * (1.0 / 63.0)
        inv = lax.rsqrt(var + 1e-5)
        h0 = xd0 + (o0 - m) * inv
        h1 = xd1 + (o1 - m) * inv
        o_ref[0, :, :] = jnp.where(h0 >= 0, h0, 0.01 * h0)
        o_ref[1, :, :] = jnp.where(h1 >= 0, h1, 0.01 * h1)

    return pl.pallas_call(
        body,
        grid=(_N3P // rb,),
        in_specs=[
            pl.BlockSpec((rb, 64), lambda i: (i, 0)),
            pl.BlockSpec((rb, 64), lambda i: (i, 0)),
            pl.BlockSpec((rb, 64), lambda i: (i, 0)),
            pl.BlockSpec((32, 32), lambda i: (0, 0)),
            pl.BlockSpec((32, 32), lambda i: (0, 0)),
            pl.BlockSpec((1, 32), lambda i: (0, 0)),
        ],
        out_specs=pl.BlockSpec((2, rb, 32), lambda i: (0, i, 0)),
        out_shape=jax.ShapeDtypeStruct((2, _N3P, 32), jnp.float32),
    )(xdf, p0, p1, ca, cb, cbias)


def _sc_resgather(table, res_pad, f, fw, ndstp, pw, ch):
    niter = pw // ch

    @functools.partial(
        pl.kernel,
        out_type=(jax.ShapeDtypeStruct((ndstp, fw), jnp.float32),
                  jax.ShapeDtypeStruct((ndstp, 16), jnp.float32)),
        mesh=_MESH,
        scratch_types=[
            pltpu.VMEM((ch,), jnp.int32),
            pltpu.VMEM((ch, f), jnp.float32),
            pltpu.VMEM((ch, fw), jnp.float32),
            pltpu.VMEM((ch, 16), jnp.float32),
            pltpu.SemaphoreType.DMA,
        ],
        compiler_params=_SC_PARAMS,
    )
    def k(table_ref, res_ref, feats_out, scal_out, idx_v, rows_v, fv, sv, sem):
        wid = lax.axis_index("s") * 2 + lax.axis_index("c")

        def chunk(j, carry):
            base = wid * pw + j * ch
            pltpu.sync_copy(res_ref.at[pl.ds(base, ch)], idx_v)
            pltpu.async_copy(table_ref.at[idx_v], rows_v, sem).wait()

            def row(e, c2):
                for kk in range(fw // 16):
                    fv[e, pl.ds(kk * 16, 16)] = rows_v[e, pl.ds(kk * 16, 16)]
                sv[e, pl.ds(0, 16)] = rows_v[e, pl.ds(fw, 16)]
                return c2

            lax.fori_loop(0, ch, row, 0)
            pltpu.sync_copy(fv, feats_out.at[pl.ds(base, ch)])
            pltpu.sync_copy(sv, scal_out.at[pl.ds(base, ch)])
            return carry

        lax.fori_loop(0, niter, chunk, 0)

    return k(table, res_pad)


def _sc_edges(table, dscal, srcp, dstp, wp, consts, f, fw, c, ndstp, epw, zch):
    mw = 2 * c
    niter = epw // _ECH
    pt = ndstp // 16
    nz = pt // zch

    @functools.partial(
        pl.kernel,
        out_type=jax.ShapeDtypeStruct((2 * ndstp, mw), jnp.float32),
        mesh=_MESH,
        scratch_types=[
            pltpu.VMEM((_ECH,), jnp.int32),
            pltpu.VMEM((_ECH,), jnp.int32),
            pltpu.VMEM((_ECH,), jnp.int32),
            pltpu.VMEM((_ECH,), jnp.int32),
            pltpu.VMEM((_ECH,), jnp.float32),
            pltpu.VMEM((_ECH,), jnp.float32),
            pltpu.VMEM((_ECH, f), jnp.float32),
            pltpu.VMEM((_ECH, f), jnp.float32),
            pltpu.VMEM((_ECH, 16), jnp.float32),
            pltpu.VMEM((_ECH, 16), jnp.float32),
            pltpu.VMEM((_ECH, mw), jnp.float32),
            pltpu.VMEM((4, 16), jnp.float32),
            pltpu.VMEM((zch, mw), jnp.float32),
            pltpu.VMEM_SHARED((ndstp, mw), jnp.float32),
            pltpu.SemaphoreType.DMA,
            pltpu.SemaphoreType.DMA,
            pltpu.SemaphoreType.DMA,
            pltpu.SemaphoreType.DMA,
        ],
        compiler_params=_SC_PARAMS,
    )
    def k(table_ref, dscal_ref, src_ref, dst_ref, w_ref, c_ref, part_out,
          sidx0, sidx1, didx0, didx1, wv0, wv1, rows0, rows1, drows0, drows1,
          msg, cv, zbuf, aggr, seml0, seml1, semg0, semg1):
        sidx = (sidx0, sidx1)
        didx = (didx0, didx1)
        wv = (wv0, wv1)
        rows = (rows0, rows1)
        drows = (drows0, drows1)
        cid = lax.axis_index("c")
        sid = lax.axis_index("s")
        wid = sid * 2 + cid
        pltpu.sync_copy(c_ref, cv)

        def col16(v):
            return jnp.full((16,), v, jnp.int32)

        zero16 = jnp.zeros((16,), jnp.float32)

        def zrow(e, carry):
            for kk in range(mw // 16):
                zbuf[e, pl.ds(kk * 16, 16)] = zero16
            return carry

        lax.fori_loop(0, zch, zrow, 0)
        for j in range(nz):
            pltpu.sync_copy(zbuf, aggr.at[pl.ds(sid * pt + j * zch, zch)])
        plsc.subcore_barrier()

        ce = cv[2, pl.ds(0, 16)][0]
        wrows = [cv[kk, pl.ds(0, 16)] for kk in range(c // 16)]
        iot = lax.iota(jnp.int32, 16)
        colq0 = col16(fw)
        colq1 = col16(fw + 1)
        colk0 = col16(2)
        colk1 = col16(3)

        def issue_l(base, b, sem):
            pltpu.async_copy(src_ref.at[pl.ds(base, _ECH)], sidx[b], sem)
            pltpu.async_copy(dst_ref.at[pl.ds(base, _ECH)], didx[b], sem)
            pltpu.async_copy(w_ref.at[pl.ds(base, _ECH)], wv[b], sem)

        def wait_l(b, sem):
            pltpu.make_async_copy(src_ref.at[pl.ds(0, _ECH)], sidx[b], sem).wait()
            pltpu.make_async_copy(dst_ref.at[pl.ds(0, _ECH)], didx[b], sem).wait()
            pltpu.make_async_copy(w_ref.at[pl.ds(0, _ECH)], wv[b], sem).wait()

        def issue_g(b, sem):
            pltpu.async_copy(table_ref.at[sidx[b]], rows[b], sem)
            pltpu.async_copy(dscal_ref.at[didx[b]], drows[b], sem)

        def wait_g(b, sem):
            pltpu.make_async_copy(table_ref.at[sidx[b]], rows[b], sem).wait()
            pltpu.make_async_copy(dscal_ref.at[didx[b]], drows[b], sem).wait()

        def compute(b):
            rows_b = rows[b]
            drows_b = drows[b]
            wv_b = wv[b]

            def group(g, c2):
                rid = iot + g * 16
                sq0 = plsc.load_gather(rows_b, [rid, colq0])
                sq1 = plsc.load_gather(rows_b, [rid, colq1])
                sk0 = plsc.load_gather(drows_b, [rid, colk0])
                sk1 = plsc.load_gather(drows_b, [rid, colk1])
                wg = wv_b[pl.ds(g * 16, 16)]
                cwg = ce * wg
                a0 = _sigmoid(sq0 + sk0 + cwg)
                a1 = _sigmoid(sq1 + sk1 + cwg)
                for kk in range(c // 16):
                    wrow = wrows[kk]
                    for cc in range(16):
                        col = kk * 16 + cc
                        gcol = _sigmoid(wg * wrow[cc])
                        x0 = plsc.load_gather(rows_b, [rid, col16(col)])
                        x1 = plsc.load_gather(rows_b, [rid, col16(c + col)])
                        plsc.store_scatter(msg, [rid, col16(col)], (a0 * gcol) * x0)
                        plsc.store_scatter(msg, [rid, col16(c + col)], (a1 * gcol) * x1)
                return c2

            lax.fori_loop(0, _ECH // 16, group, 0)
            pltpu.sync_copy(msg, aggr.at[didx[b]], add=True)

        start = wid * epw
        issue_l(start, 0, seml0)
        issue_l(start + _ECH, 1, seml1)
        wait_l(0, seml0)
        issue_g(0, semg0)

        def pair(jj, carry):
            base = start + 2 * jj * _ECH
            wait_g(0, semg0)
            wait_l(1, seml1)
            issue_g(1, semg1)
            compute(0)
            issue_l(base + 2 * _ECH, 0, seml0)
            wait_g(1, semg1)
            wait_l(0, seml0)
            issue_g(0, semg0)
            compute(1)
            issue_l(base + 3 * _ECH, 1, seml1)
            return carry

        lax.fori_loop(0, niter // 2 - 1, pair, 0)
        wait_g(0, semg0)
        wait_l(1, seml1)
        issue_g(1, semg1)
        compute(0)
        wait_g(1, semg1)
        compute(1)
        plsc.subcore_barrier()

        for j in range(nz):
            off = sid * pt + j * zch
            pltpu.sync_copy(aggr.at[pl.ds(off, zch)], zbuf)
            pltpu.sync_copy(zbuf, part_out.at[pl.ds(cid * ndstp + off, zch)])

    return k(table, dscal, srcp, dstp, wp, consts)


def kernel(X, edge_index_0, edge_weight_0, edge_index_1, edge_weight_1,
           res_n_id_0, res_n_id_1, size0_dst, size1_dst,
           w_n1, w_e1, q1, k1, att_w1, att_b1, cat_w1, cat_b1,
           w_n2, w_e2, q2, k2, att_w2, att_b2, cat_w2, cat_b2):
    f32 = jnp.float32

    vq1 = w_n1 @ (q1 @ att_w1[0:16, 0])
    vk1 = w_n1 @ (k1 @ att_w1[16:32, 0])
    ce1 = w_e1[0] @ att_w1[32:48, 0]
    w0 = (jnp.zeros((32, 48), f32)
          .at[:, 0:16].set(w_n1).at[:, 32].set(vq1).at[:, 34].set(vk1))
    w1 = (jnp.zeros((32, 48), f32)
          .at[:, 16:32].set(w_n1).at[:, 33].set(vq1).at[:, 35].set(vk1))
    bias1 = jnp.zeros((1, 48), f32).at[0, 34].set(att_b1[0]).at[0, 35].set(att_b1[0])

    xs1 = _tc_proj1(X[0], X[1], w0, w1, bias1)

    res0p = jnp.pad(res_n_id_0.astype(jnp.int32), (0, _N2P - _N2))
    xd1, dscal1 = _sc_resgather(xs1, res0p, 48, 32, _N2P, 784, 112)

    pad_e1 = _NW * _E1PW - _E1
    src0p = jnp.pad(edge_index_0[0].astype(jnp.int32), (0, pad_e1))
    dst0p = jnp.pad(edge_index_0[1].astype(jnp.int32), (0, pad_e1),
                    constant_values=_N2P - 1)
    ew0p = jnp.pad(edge_weight_0, (0, pad_e1))
    consts1 = jnp.zeros((4, 16), f32).at[0].set(w_e1[0]).at[2, 0].set(ce1)
    part1 = _sc_edges(xs1, dscal1, src0p, dst0p, ew0p, consts1,
                      48, 32, 16, _N2P, _E1PW, 112)

    vq2 = w_n2 @ (q2 @ att_w2[0:32, 0])
    vk2 = w_n2 @ (k2 @ att_w2[32:64, 0])
    ce2 = w_e2[0] @ att_w2[64:96, 0]
    m0 = (jnp.zeros((16, 80), f32)
          .at[:, 0:32].set(w_n2).at[:, 64].set(vq2).at[:, 66].set(vk2))
    m1 = (jnp.zeros((16, 80), f32)
          .at[:, 32:64].set(w_n2).at[:, 65].set(vq2).at[:, 67].set(vk2))
    bias2 = jnp.zeros((1, 80), f32).at[0, 66].set(att_b2[0]).at[0, 67].set(att_b2[0])

    xs2 = _tc_update1(xd1, part1[0:_N2P], part1[_N2P:2 * _N2P],
                      cat_w1[0:16], cat_w1[16:32], cat_b1[None, :],
                      m0, m1, bias2)

    res1p = jnp.pad(res_n_id_1.astype(jnp.int32), (0, _N3P - _N3))
    xd2, dscal2 = _sc_resgather(xs2, res1p, 80, 64, _N3P, 392, 56)

    pad_e2 = _NW * _E2PW - _E2
    src1p = jnp.pad(edge_index_1[0].astype(jnp.int32), (0, pad_e2))
    dst1p = jnp.pad(edge_index_1[1].astype(jnp.int32), (0, pad_e2),
                    constant_values=_N3P - 1)
    ew1p = jnp.pad(edge_weight_1, (0, pad_e2))
    consts2 = (jnp.zeros((4, 16), f32)
               .at[0].set(w_e2[0, 0:16]).at[1].set(w_e2[0, 16:32])
               .at[2, 0].set(ce2))
    part2 = _sc_edges(xs2, dscal2, src1p, dst1p, ew1p, consts2,
                      80, 64, 32, _N3P, _E2PW, 56)

    outp = _tc_final(xd2, part2[0:_N3P], part2[_N3P:2 * _N3P],
                     cat_w2[0:32], cat_w2[32:64], cat_b2[None, :])
    return outp[:, 0:_N3, :]


# speedup vs baseline: 45.2054x; 1.0352x over previous
import functools

import jax
import jax.numpy as jnp
from jax import lax
from jax.experimental import pallas as pl
from jax.experimental.pallas import tpu as pltpu
from jax.experimental.pallas import tpu_sc as plsc

_N1, _N2, _N3 = 50000, 25000, 12500
_E1, _E2 = 800000, 400000

_NW = 32
_N2P = 25088
_N3P = 12544
_E1PW, _E2PW = 25088, 12544
_ECH = 128

_MESH = plsc.VectorSubcoreMesh(
    core_axis_name="c", subcore_axis_name="s", num_cores=2, num_subcores=16)
_SC_PARAMS = pltpu.CompilerParams(
    use_tc_tiling_on_sc=False, needs_layout_passes=False)


def _sigmoid(x):
    return 1.0 / (1.0 + jnp.exp(-x))


def _tc_proj1(x0, x1, w0, w1, bias):
    rb = 400

    def body(x0_ref, x1_ref, w0_ref, w1_ref, b_ref, o_ref):
        o_ref[...] = (
            jnp.dot(x0_ref[...], w0_ref[...], preferred_element_type=jnp.float32)
            + jnp.dot(x1_ref[...], w1_ref[...], preferred_element_type=jnp.float32)
            + b_ref[...])

    return pl.pallas_call(
        body,
        grid=(_N1 // rb,),
        in_specs=[
            pl.BlockSpec((rb, 32), lambda i: (i, 0)),
            pl.BlockSpec((rb, 32), lambda i: (i, 0)),
            pl.BlockSpec((32, 48), lambda i: (0, 0)),
            pl.BlockSpec((32, 48), lambda i: (0, 0)),
            pl.BlockSpec((1, 48), lambda i: (0, 0)),
        ],
        out_specs=pl.BlockSpec((rb, 48), lambda i: (i, 0)),
        out_shape=jax.ShapeDtypeStruct((_N1, 48), jnp.float32),
    )(x0, x1, w0, w1, bias)


def _tc_update1(xdf, p0, p1, ca, cb, cbias, m0, m1, b2):
    rb = 256

    def body(xdf_ref, p0_ref, p1_ref, ca_ref, cb_ref, cbias_ref,
             m0_ref, m1_ref, b2_ref, o_ref):
        xd0 = xdf_ref[:, 0:16]
        xd1 = xdf_ref[:, 16:32]
        a0 = p0_ref[:, 0:16] + p1_ref[:, 0:16]
        a1 = p0_ref[:, 16:32] + p1_ref[:, 16:32]
        o0 = (jnp.dot(xd0, ca_ref[...], preferred_element_type=jnp.float32)
              + jnp.dot(a0, cb_ref[...], preferred_element_type=jnp.float32)
              + cbias_ref[...])
        o1 = (jnp.dot(xd1, ca_ref[...], preferred_element_type=jnp.float32)
              + jnp.dot(a1, cb_ref[...], preferred_element_type=jnp.float32)
              + cbias_ref[...])
        s = jnp.sum(o0, axis=1, keepdims=True) + jnp.sum(o1, axis=1, keepdims=True)
        m = s * (1.0 / 32.0)
        ss = (jnp.sum(o0 * o0, axis=1, keepdims=True)
              + jnp.sum(o1 * o1, axis=1, keepdims=True))
        var = (ss - 32.0 * m * m) * (1.0 / 31.0)
        inv = lax.rsqrt(var + 1e-5)
        h0 = xd0 + (o0 - m) * inv
        h1 = xd1 + (o1 - m) * inv
        h0 = jnp.where(h0 >= 0, h0, 0.01 * h0)
        h1 = jnp.where(h1 >= 0, h1, 0.01 * h1)
        o_ref[...] = (
            jnp.dot(h0, m0_ref[...], preferred_element_type=jnp.float32)
            + jnp.dot(h1, m1_ref[...], preferred_element_type=jnp.float32)
            + b2_ref[...])

    return pl.pallas_call(
        body,
        grid=(_N2P // rb,),
        in_specs=[
            pl.BlockSpec((rb, 32), lambda i: (i, 0)),
            pl.BlockSpec((rb, 32), lambda i: (i, 0)),
            pl.BlockSpec((rb, 32), lambda i: (i, 0)),
            pl.BlockSpec((16, 16), lambda i: (0, 0)),
            pl.BlockSpec((16, 16), lambda i: (0, 0)),
            pl.BlockSpec((1, 16), lambda i: (0, 0)),
            pl.BlockSpec((16, 80), lambda i: (0, 0)),
            pl.BlockSpec((16, 80), lambda i: (0, 0)),
            pl.BlockSpec((1, 80), lambda i: (0, 0)),
        ],
        out_specs=pl.BlockSpec((rb, 80), lambda i: (i, 0)),
        out_shape=jax.ShapeDtypeStruct((_N2P, 80), jnp.float32),
    )(xdf, p0, p1, ca, cb, cbias, m0, m1, b2)


def _tc_final(xdf, p0, p1, ca, cb, cbias):
    rb = 256

    def body(xdf_ref, p0_ref, p1_ref, ca_ref, cb_ref, cbias_ref, o_ref):
        xd0 = xdf_ref[:, 0:32]
        xd1 = xdf_ref[:, 32:64]
        a0 = p0_ref[:, 0:32] + p1_ref[:, 0:32]
        a1 = p0_ref[:, 32:64] + p1_ref[:, 32:64]
        o0 = (jnp.dot(xd0, ca_ref[...], preferred_element_type=jnp.float32)
              + jnp.dot(a0, cb_ref[...], preferred_element_type=jnp.float32)
              + cbias_ref[...])
        o1 = (jnp.dot(xd1, ca_ref[...], preferred_element_type=jnp.float32)
              + jnp.dot(a1, cb_ref[...], preferred_element_type=jnp.float32)
              + cbias_ref[...])
        s = jnp.sum(o0, axis=1, keepdims=True) + jnp.sum(o1, axis=1, keepdims=True)
        m = s * (1.0 / 64.0)
        ss = (jnp.sum(o0 * o0, axis=1, keepdims=True)
              + jnp.sum(o1 * o1, axis=1, keepdims=True))
        var = (ss - 64.0 * m * m) * (1.0 / 63.0)
        inv = lax.rsqrt(var + 1e-5)
        h0 = xd0 + (o0 - m) * inv
        h1 = xd1 + (o1 - m) * inv
        o_ref[0, :, :] = jnp.where(h0 >= 0, h0, 0.01 * h0)
        o_ref[1, :, :] = jnp.where(h1 >= 0, h1, 0.01 * h1)

    return pl.pallas_call(
        body,
        grid=(_N3P // rb,),
        in_specs=[
            pl.BlockSpec((rb, 64), lambda i: (i, 0)),
            pl.BlockSpec((rb, 64), lambda i: (i, 0)),
            pl.BlockSpec((rb, 64), lambda i: (i, 0)),
            pl.BlockSpec((32, 32), lambda i: (0, 0)),
            pl.BlockSpec((32, 32), lambda i: (0, 0)),
            pl.BlockSpec((1, 32), lambda i: (0, 0)),
        ],
        out_specs=pl.BlockSpec((2, rb, 32), lambda i: (0, i, 0)),
        out_shape=jax.ShapeDtypeStruct((2, _N3P, 32), jnp.float32),
    )(xdf, p0, p1, ca, cb, cbias)


def _sc_resgather(table, res_pad, f, fw, ndstp, pw, ch):
    niter = pw // ch

    @functools.partial(
        pl.kernel,
        out_type=(jax.ShapeDtypeStruct((ndstp, fw), jnp.float32),
                  jax.ShapeDtypeStruct((ndstp, 16), jnp.float32)),
        mesh=_MESH,
        scratch_types=[
            pltpu.VMEM((ch,), jnp.int32),
            pltpu.VMEM((ch, f), jnp.float32),
            pltpu.VMEM((ch, fw), jnp.float32),
            pltpu.VMEM((ch, 16), jnp.float32),
            pltpu.SemaphoreType.DMA,
        ],
        compiler_params=_SC_PARAMS,
    )
    def k(table_ref, res_ref, feats_out, scal_out, idx_v, rows_v, fv, sv, sem):
        wid = lax.axis_index("s") * 2 + lax.axis_index("c")

        def chunk(j, carry):
            base = wid * pw + j * ch
            pltpu.sync_copy(res_ref.at[pl.ds(base, ch)], idx_v)
            pltpu.async_copy(table_ref.at[idx_v], rows_v, sem).wait()

            def row(e, c2):
                for kk in range(fw // 16):
                    fv[e, pl.ds(kk * 16, 16)] = rows_v[e, pl.ds(kk * 16, 16)]
                sv[e, pl.ds(0, 16)] = rows_v[e, pl.ds(fw, 16)]
                return c2

            lax.fori_loop(0, ch, row, 0)
            pltpu.sync_copy(fv, feats_out.at[pl.ds(base, ch)])
            pltpu.sync_copy(sv, scal_out.at[pl.ds(base, ch)])
            return carry

        lax.fori_loop(0, niter, chunk, 0)

    return k(table, res_pad)


def _sc_edges(table, dscal, srcp, dstp, wp, consts, f, fw, c, ndstp, epw, zch):
    mw = 2 * c
    niter = epw // _ECH
    pt = ndstp // 16
    nz = pt // zch

    @functools.partial(
        pl.kernel,
        out_type=jax.ShapeDtypeStruct((2 * ndstp, mw), jnp.float32),
        mesh=_MESH,
        scratch_types=[
            pltpu.VMEM((_ECH,), jnp.int32),
            pltpu.VMEM((_ECH,), jnp.int32),
            pltpu.VMEM((_ECH,), jnp.int32),
            pltpu.VMEM((_ECH,), jnp.int32),
            pltpu.VMEM((_ECH,), jnp.float32),
            pltpu.VMEM((_ECH,), jnp.float32),
            pltpu.VMEM((_ECH, f), jnp.float32),
            pltpu.VMEM((_ECH, f), jnp.float32),
            pltpu.VMEM((_ECH, 16), jnp.float32),
            pltpu.VMEM((_ECH, 16), jnp.float32),
            pltpu.VMEM((_ECH,), jnp.int32),
            pltpu.VMEM((_ECH,), jnp.int32),
            pltpu.VMEM((_ECH, mw), jnp.float32),
            pltpu.VMEM((_ECH, mw), jnp.float32),
            pltpu.VMEM((4, 16), jnp.float32),
            pltpu.VMEM((zch, mw), jnp.float32),
            pltpu.VMEM_SHARED((ndstp, mw), jnp.float32),
            pltpu.SemaphoreType.DMA,
            pltpu.SemaphoreType.DMA,
            pltpu.SemaphoreType.DMA,
            pltpu.SemaphoreType.DMA,
            pltpu.SemaphoreType.DMA,
            pltpu.SemaphoreType.DMA,
            pltpu.SemaphoreType.DMA,
        ],
        compiler_params=_SC_PARAMS,
    )
    def k(table_ref, dscal_ref, src_ref, dst_ref, w_ref, c_ref, part_out,
          sidx0, sidx1, didx0, didx1, wv0, wv1, rows0, rows1, drows0, drows1,
          didxs0, didxs1, msg0, msg1, cv, zbuf, aggr,
          seml0, seml1, semg0, semg1, seml2, sems0, sems1):
        sidx = (sidx0, sidx1)
        didx = (didx0, didx1)
        wv = (wv0, wv1)
        rows = (rows0, rows1)
        drows = (drows0, drows1)
        didxs = (didxs0, didxs1)
        msg = (msg0, msg1)
        sems = (sems0, sems1)
        cid = lax.axis_index("c")
        sid = lax.axis_index("s")
        wid = sid * 2 + cid
        pltpu.sync_copy(c_ref, cv)

        def col16(v):
            return jnp.full((16,), v, jnp.int32)

        zero16 = jnp.zeros((16,), jnp.float32)

        def zrow(e, carry):
            for kk in range(mw // 16):
                zbuf[e, pl.ds(kk * 16, 16)] = zero16
            return carry

        lax.fori_loop(0, zch, zrow, 0)
        for j in range(nz):
            pltpu.sync_copy(zbuf, aggr.at[pl.ds(sid * pt + j * zch, zch)])
        plsc.subcore_barrier()

        ce = cv[2, pl.ds(0, 16)][0]
        wrows = [cv[kk, pl.ds(0, 16)] for kk in range(c // 16)]
        iot = lax.iota(jnp.int32, 16)
        colq0 = col16(fw)
        colq1 = col16(fw + 1)
        colk0 = col16(2)
        colk1 = col16(3)

        def issue_l(base, b, sem):
            pltpu.async_copy(src_ref.at[pl.ds(base, _ECH)], sidx[b], sem)
            pltpu.async_copy(dst_ref.at[pl.ds(base, _ECH)], didx[b], sem)
            pltpu.async_copy(w_ref.at[pl.ds(base, _ECH)], wv[b], sem)

        def wait_l(b, sem):
            pltpu.make_async_copy(src_ref.at[pl.ds(0, _ECH)], sidx[b], sem).wait()
            pltpu.make_async_copy(dst_ref.at[pl.ds(0, _ECH)], didx[b], sem).wait()
            pltpu.make_async_copy(w_ref.at[pl.ds(0, _ECH)], wv[b], sem).wait()

        def issue_g(b, sem):
            pltpu.async_copy(table_ref.at[sidx[b]], rows[b], sem)
            pltpu.async_copy(dscal_ref.at[didx[b]], drows[b], sem)

        def wait_g(b, sem):
            pltpu.make_async_copy(table_ref.at[sidx[b]], rows[b], sem).wait()
            pltpu.make_async_copy(dscal_ref.at[didx[b]], drows[b], sem).wait()

        def compute(b):
            rows_b = rows[b]
            drows_b = drows[b]
            wv_b = wv[b]
            msg_b = msg[b]

            def group(g, c2):
                rid = iot + g * 16
                sq0 = plsc.load_gather(rows_b, [rid, colq0])
                sq1 = plsc.load_gather(rows_b, [rid, colq1])
                sk0 = plsc.load_gather(drows_b, [rid, colk0])
                sk1 = plsc.load_gather(drows_b, [rid, colk1])
                wg = wv_b[pl.ds(g * 16, 16)]
                cwg = ce * wg
                a0 = _sigmoid(sq0 + sk0 + cwg)
                a1 = _sigmoid(sq1 + sk1 + cwg)
                for kk in range(c // 16):
                    wrow = wrows[kk]
                    for cc in range(16):
                        col = kk * 16 + cc
                        gcol = _sigmoid(wg * wrow[cc])
                        x0 = plsc.load_gather(rows_b, [rid, col16(col)])
                        x1 = plsc.load_gather(rows_b, [rid, col16(c + col)])
                        plsc.store_scatter(msg_b, [rid, col16(col)], (a0 * gcol) * x0)
                        plsc.store_scatter(msg_b, [rid, col16(c + col)], (a1 * gcol) * x1)
                return c2

            lax.fori_loop(0, _ECH // 16, group, 0)

        def issue_l2(base, b):
            pltpu.async_copy(dst_ref.at[pl.ds(base, _ECH)], didxs[b], seml2)

        def wait_l2(b):
            pltpu.make_async_copy(dst_ref.at[pl.ds(0, _ECH)], didxs[b], seml2).wait()

        def start_s(b):
            pltpu.make_async_copy(msg[b], aggr.at[didxs[b]], sems[b]).start(add=True)

        def wait_s(b):
            pltpu.make_async_copy(msg[b], aggr.at[didxs[b]], sems[b]).wait()

        start = wid * epw
        issue_l(start, 0, seml0)
        issue_l(start + _ECH, 1, seml1)
        wait_l(0, seml0)
        issue_g(0, semg0)

        def half(base, b, jj):
            wait_g(b, (semg0, semg1)[b])
            wait_l(1 - b, (seml0, seml1)[1 - b])
            issue_g(1 - b, (semg0, semg1)[1 - b])

            @pl.when(jj > 0)
            def _():
                wait_s(b)

            issue_l2(base, b)
            compute(b)
            wait_l2(b)
            start_s(b)
            issue_l(base + 2 * _ECH, b, (seml0, seml1)[b])

        def pair(jj, carry):
            base = start + 2 * jj * _ECH
            half(base, 0, jj)
            half(base + _ECH, 1, jj)
            return carry

        lax.fori_loop(0, niter // 2 - 1, pair, 0)
        base = start + (niter - 2) * _ECH
        wait_g(0, semg0)
        wait_l(1, seml1)
        issue_g(1, semg1)
        wait_s(0)
        issue_l2(base, 0)
        compute(0)
        wait_l2(0)
        start_s(0)
        wait_g(1, semg1)
        wait_s(1)
        issue_l2(base + _ECH, 1)
        compute(1)
        wait_l2(1)
        start_s(1)
        wait_s(0)
        wait_s(1)
        plsc.subcore_barrier()

        for j in range(nz):
            off = sid * pt + j * zch
            pltpu.sync_copy(aggr.at[pl.ds(off, zch)], zbuf)
            pltpu.sync_copy(zbuf, part_out.at[pl.ds(cid * ndstp + off, zch)])

    return k(table, dscal, srcp, dstp, wp, consts)


def kernel(X, edge_index_0, edge_weight_0, edge_index_1, edge_weight_1,
           res_n_id_0, res_n_id_1, size0_dst, size1_dst,
           w_n1, w_e1, q1, k1, att_w1, att_b1, cat_w1, cat_b1,
           w_n2, w_e2, q2, k2, att_w2, att_b2, cat_w2, cat_b2):
    f32 = jnp.float32

    vq1 = w_n1 @ (q1 @ att_w1[0:16, 0])
    vk1 = w_n1 @ (k1 @ att_w1[16:32, 0])
    ce1 = w_e1[0] @ att_w1[32:48, 0]
    w0 = (jnp.zeros((32, 48), f32)
          .at[:, 0:16].set(w_n1).at[:, 32].set(vq1).at[:, 34].set(vk1))
    w1 = (jnp.zeros((32, 48), f32)
          .at[:, 16:32].set(w_n1).at[:, 33].set(vq1).at[:, 35].set(vk1))
    bias1 = jnp.zeros((1, 48), f32).at[0, 34].set(att_b1[0]).at[0, 35].set(att_b1[0])

    xs1 = _tc_proj1(X[0], X[1], w0, w1, bias1)

    res0p = jnp.pad(res_n_id_0.astype(jnp.int32), (0, _N2P - _N2))
    xd1, dscal1 = _sc_resgather(xs1, res0p, 48, 32, _N2P, 784, 112)

    pad_e1 = _NW * _E1PW - _E1
    src0p = jnp.pad(edge_index_0[0].astype(jnp.int32), (0, pad_e1))
    dst0p = jnp.pad(edge_index_0[1].astype(jnp.int32), (0, pad_e1),
                    constant_values=_N2P - 1)
    ew0p = jnp.pad(edge_weight_0, (0, pad_e1))
    consts1 = jnp.zeros((4, 16), f32).at[0].set(w_e1[0]).at[2, 0].set(ce1)
    part1 = _sc_edges(xs1, dscal1, src0p, dst0p, ew0p, consts1,
                      48, 32, 16, _N2P, _E1PW, 112)

    vq2 = w_n2 @ (q2 @ att_w2[0:32, 0])
    vk2 = w_n2 @ (k2 @ att_w2[32:64, 0])
    ce2 = w_e2[0] @ att_w2[64:96, 0]
    m0 = (jnp.zeros((16, 80), f32)
          .at[:, 0:32].set(w_n2).at[:, 64].set(vq2).at[:, 66].set(vk2))
    m1 = (jnp.zeros((16, 80), f32)
          .at[:, 32:64].set(w_n2).at[:, 65].set(vq2).at[:, 67].set(vk2))
    bias2 = jnp.zeros((1, 80), f32).at[0, 66].set(att_b2[0]).at[0, 67].set(att_b2[0])

    xs2 = _tc_update1(xd1, part1[0:_N2P], part1[_N2P:2 * _N2P],
                      cat_w1[0:16], cat_w1[16:32], cat_b1[None, :],
                      m0, m1, bias2)

    res1p = jnp.pad(res_n_id_1.astype(jnp.int32), (0, _N3P - _N3))
    xd2, dscal2 = _sc_resgather(xs2, res1p, 80, 64, _N3P, 392, 56)

    pad_e2 = _NW * _E2PW - _E2
    src1p = jnp.pad(edge_index_1[0].astype(jnp.int32), (0, pad_e2))
    dst1p = jnp.pad(edge_index_1[1].astype(jnp.int32), (0, pad_e2),
                    constant_values=_N3P - 1)
    ew1p = jnp.pad(edge_weight_1, (0, pad_e2))
    consts2 = (jnp.zeros((4, 16), f32)
               .at[0].set(w_e2[0, 0:16]).at[1].set(w_e2[0, 16:32])
               .at[2, 0].set(ce2))
    part2 = _sc_edges(xs2, dscal2, src1p, dst1p, ew1p, consts2,
                      80, 64, 32, _N3P, _E2PW, 56)

    outp = _tc_final(xd2, part2[0:_N3P], part2[_N3P:2 * _N3P],
                     cat_w2[0:32], cat_w2[32:64], cat_b2[None, :])
    return outp[:, 0:_N3, :]
